# Initial kernel scaffold; baseline (speedup 1.0000x reference)
import jax
import jax.numpy as jnp
from jax.experimental import pallas as pl


def kernel(master_feats, bond_feats, global_feats, edge_index_bond, src_global, dst_global, W1, b1, W2, b2, W3, b3):
    raise NotImplementedError("write your pallas kernel here")


# speedup vs baseline: 9.2624x; 9.2624x over previous
import functools

import jax
import jax.numpy as jnp
from jax import lax
from jax.experimental import pallas as pl
from jax.experimental.pallas import tpu as pltpu
from jax.experimental.pallas import tpu_sc as plsc

N_ATOM = 10000
N_BOND = 10000
N_GLOBAL = 64
E_BOND = 320000
E_GLOBAL = 10000
D = 128

NC = 2
NS = 16
NW = NC * NS

PAD_N = 10240
ZROWS = PAD_N // NS
DROWS = PAD_N // D
L = 16

CH = 80
GRP = 25
NGRP_A = 5
NGRP_B = 1
GRP_B = 4
NBUF = 2
E_GLOBAL_P = NW * GRP_B * CH

BLK = 2048


def _sc_aggregate(tab_a, tab_b, sd_a, sd_b, zeros_sl):
    mesh = plsc.VectorSubcoreMesh(
        core_axis_name="c", subcore_axis_name="s", num_cores=NC, num_subcores=NS
    )

    @functools.partial(
        pl.kernel,
        out_type=[
            jax.ShapeDtypeStruct((NC, PAD_N, D), jnp.float32),
            jax.ShapeDtypeStruct((NC, PAD_N, D), jnp.float32),
            jax.ShapeDtypeStruct((NC, NS, DROWS, D), jnp.float32),
            jax.ShapeDtypeStruct((NC, NS, DROWS, D), jnp.float32),
        ],
        mesh=mesh,
        compiler_params=pltpu.CompilerParams(needs_layout_passes=False),
        scratch_types=[
            pltpu.VMEM_SHARED((PAD_N, D), jnp.float32),
            pltpu.VMEM((GRP, CH), jnp.int32),
            pltpu.VMEM((GRP, CH), jnp.int32),
            *[pltpu.VMEM((CH, D), jnp.float32) for _ in range(NBUF)],
            pltpu.VMEM((DROWS, D), jnp.float32),
            *[pltpu.SemaphoreType.DMA for _ in range(NBUF)],
            pltpu.SemaphoreType.DMA,
            pltpu.SemaphoreType.DMA,
        ],
    )
    def body(tab_a_h, tab_b_h, sd_a_h, sd_b_h, zz_h,
             out_a_h, out_b_h, deg_a_h, deg_b_h,
             acc, srcv, dstv, *rest):
        rowbufs = rest[:NBUF]
        ldeg = rest[NBUF]
        gsems_l = rest[NBUF + 1:NBUF + 1 + NBUF]
        sems0, sems1 = rest[NBUF + 1 + NBUF:]
        c = lax.axis_index("c")
        s = lax.axis_index("s")
        wid = c * NS + s
        ones = jnp.full((L,), 1.0, jnp.float32)

        def do_phase(table_h, sd_h, out_h, deg_h, ngrp, grp):
            pltpu.sync_copy(zz_h, acc.at[pl.ds(s * ZROWS, ZROWS)])
            pltpu.sync_copy(zz_h.at[pl.ds(0, DROWS)], ldeg)
            plsc.subcore_barrier()

            bufs = tuple(rowbufs)
            gsems = tuple(gsems_l)
            ssems = (sems0, sems1)

            def group(g, carry):
                pltpu.sync_copy(sd_h.at[0, wid, g], srcv.at[pl.ds(0, grp)])
                pltpu.sync_copy(sd_h.at[1, wid, g], dstv.at[pl.ds(0, grp)])
                gat = [None, None]
                scat = [None, None]
                gat[0] = pltpu.async_copy(table_h.at[srcv.at[0]], bufs[0], gsems[0])
                for i in range(grp):
                    p = i % 2
                    for j in range(CH // L):
                        dd = dstv[i, pl.ds(j * L, L)]
                        plsc.addupdate_scatter(
                            ldeg,
                            [lax.shift_right_logical(dd, 7),
                             lax.bitwise_and(dd, 127)], ones)
                    if i + 1 < grp:
                        if scat[1 - p] is not None:
                            scat[1 - p].wait()
                        gat[1 - p] = pltpu.async_copy(
                            table_h.at[srcv.at[i + 1]], bufs[1 - p], gsems[1 - p])
                    gat[p].wait()
                    scat[p] = pltpu.async_copy(
                        bufs[p], acc.at[dstv.at[i]], ssems[p], add=True)
                if grp >= 2 and scat[(grp - 2) % 2] is not None:
                    scat[(grp - 2) % 2].wait()
                scat[(grp - 1) % 2].wait()
                return carry

            lax.fori_loop(0, ngrp, group, 0)
            pltpu.sync_copy(ldeg, deg_h.at[c, s])
            plsc.subcore_barrier()
            pltpu.sync_copy(
                acc.at[pl.ds(s * ZROWS, ZROWS)],
                out_h.at[c, pl.ds(s * ZROWS, ZROWS)],
            )
            plsc.subcore_barrier()

        do_phase(tab_a_h, sd_a_h, out_a_h, deg_a_h, NGRP_A, GRP)
        do_phase(tab_b_h, sd_b_h, out_b_h, deg_b_h, NGRP_B, GRP_B)

    return body(tab_a, tab_b, sd_a, sd_b, zeros_sl)


def _deg_sum_body(da, db, oa, ob):
    oa[...] = jnp.sum(da[...], axis=0)
    ob[...] = jnp.sum(db[...], axis=0)


def _tc_deg_sum(deg_a, deg_b):
    return pl.pallas_call(
        _deg_sum_body,
        out_shape=[
            jax.ShapeDtypeStruct((DROWS, D), jnp.float32),
            jax.ShapeDtypeStruct((DROWS, D), jnp.float32),
        ],
    )(deg_a, deg_b)


def _softplus(x):
    mx = jnp.maximum(x, 0.0)
    return mx + jnp.log(jnp.exp(-mx) + jnp.exp(x - mx))


def _mlp_body(mf, a0, a1, g0, g1, da, dg,
              w1a, w1b, w1c, bb1, w2, bb2, w3, bb3, out):
    m1 = (a0 + a1) / jnp.maximum(da, 1.0)
    m2 = (g0 + g1) / jnp.maximum(dg, 1.0)
    h = (
        jnp.dot(mf, w1a, preferred_element_type=jnp.float32)
        + jnp.dot(m1, w1b, preferred_element_type=jnp.float32)
        + jnp.dot(m2, w1c, preferred_element_type=jnp.float32)
        + bb1
    )
    h = _softplus(h)
    h = _softplus(jnp.dot(h, w2, preferred_element_type=jnp.float32) + bb2)
    out[...] = jnp.dot(h, w3, preferred_element_type=jnp.float32) + bb3


def _tc_mlp(mfp, acc_a, acc_b, deg_a, deg_b, w1a, w1b, w1c, bb1, w2, bb2, w3, bb3):
    n_out = w3.shape[1]
    grid = (PAD_N // BLK,)
    blk_rows = lambda i: (i, 0)
    full = lambda i: (0, 0)
    slab = lambda k: pl.BlockSpec((1, BLK, D), lambda i, k=k: (k, i, 0))
    return pl.pallas_call(
        lambda mf, a0, a1, g0, g1, da, dg, *rest: _mlp_body(
            mf[...], a0[0], a1[0], g0[0], g1[0], da[...], dg[...],
            *(r[...] for r in rest[:-1]), rest[-1]
        ),
        grid=grid,
        in_specs=[
            pl.BlockSpec((BLK, D), blk_rows),
            slab(0), slab(1), slab(0), slab(1),
            pl.BlockSpec((BLK, 1), blk_rows),
            pl.BlockSpec((BLK, 1), blk_rows),
            pl.BlockSpec((D, 64), full),
            pl.BlockSpec((D, 64), full),
            pl.BlockSpec((D, 64), full),
            pl.BlockSpec((1, 64), full),
            pl.BlockSpec((64, 64), full),
            pl.BlockSpec((1, 64), full),
            pl.BlockSpec((64, n_out), full),
            pl.BlockSpec((1, n_out), full),
        ],
        out_specs=pl.BlockSpec((BLK, n_out), blk_rows),
        out_shape=jax.ShapeDtypeStruct((N_ATOM, n_out), jnp.float32),
    )(mfp, acc_a, acc_a, acc_b, acc_b, deg_a, deg_b,
      w1a, w1b, w1c, bb1, w2, bb2, w3, bb3)


def kernel(master_feats, bond_feats, global_feats, edge_index_bond,
           src_global, dst_global, W1, b1, W2, b2, W3, b3):
    f32 = jnp.float32
    i32 = jnp.int32

    sd_a = edge_index_bond.reshape(2, NW, NGRP_A, GRP, CH)

    n_pad = E_GLOBAL_P - E_GLOBAL
    sd_b = jnp.stack([
        jnp.concatenate([src_global, jnp.zeros((n_pad,), i32)]),
        jnp.concatenate([dst_global, jnp.full((n_pad,), N_ATOM, i32)]),
    ]).reshape(2, NW, NGRP_B, GRP_B, CH)

    zeros_sl = jnp.zeros((ZROWS, D), f32)

    acc_a, acc_b, deg_a, deg_b = _sc_aggregate(
        bond_feats, global_feats, sd_a, sd_b, zeros_sl
    )
    deg_a, deg_b = _tc_deg_sum(
        deg_a.reshape(NW, DROWS, D), deg_b.reshape(NW, DROWS, D)
    )
    deg_a = deg_a.reshape(PAD_N, 1)
    deg_b = deg_b.reshape(PAD_N, 1)

    w1a, w1b, w1c = W1[:D], W1[D:2 * D], W1[2 * D:]
    return _tc_mlp(
        master_feats, acc_a, acc_b, deg_a, deg_b, w1a, w1b, w1c,
        b1.reshape(1, -1), W2, b2.reshape(1, -1), W3, b3.reshape(1, -1),
    )
